# baseline (speedup 1.0000x reference)
import functools

import jax
import jax.numpy as jnp
from jax import lax
from jax.experimental import pallas as pl
from jax.experimental.pallas import tpu as pltpu, tpu_sc as plsc

B, N, E = 64, 1000, 8000
HEADS, HID = 4, 128
NC, NS = 2, 16
NW = NC * NS
PAIRS_PER_W = (2 * B) // NW
NEG = -1e30


def _sc_kernel_body(nb1_hbm, nb2_hbm, adj1_hbm, adj2_hbm, emb_hbm, c_hbm,
                    out_hbm, dst_l, src_l, comp_l, nb_l, c_v,
                    rows_ch_l, outb_v, sem_in, sem_g):
    wid = lax.axis_index("s") * NC + lax.axis_index("c")
    g = lax.shift_right_logical(wid, 4)
    iota = jnp.arange(16, dtype=jnp.int32)
    zeros16i = jnp.zeros((16,), jnp.int32)
    zerof = jnp.zeros((16,), jnp.float32)

    pltpu.sync_copy(c_hbm.at[g], c_v)

    def run_graph(nb_hbm, adj_hbm):
      base = (wid & 15) * PAIRS_PER_W

      for i in range(PAIRS_PER_W):
          b = base + i
          pltpu.async_copy(adj_hbm.at[b, 1], dst_l[i], sem_in[i])
          pltpu.async_copy(adj_hbm.at[b, 0], src_l[i], sem_in[i])
          pltpu.async_copy(nb_hbm.at[b], nb_l[i], sem_in[i])

      cnts = []
      for i in range(PAIRS_PER_W):
          b = base + i
          pltpu.make_async_copy(adj_hbm.at[b, 1], dst_l[i],
                                sem_in[i]).wait()
          pltpu.make_async_copy(adj_hbm.at[b, 0], src_l[i],
                                sem_in[i]).wait()
          pltpu.make_async_copy(nb_hbm.at[b], nb_l[i], sem_in[i]).wait()

          comp_l[i][pl.ds(0, 16)] = zeros16i

          @plsc.parallel_loop(0, E, step=16, unroll=8,
                              carry=jnp.ones((16,), jnp.int32))
          def scan_loop(ch, cntv, i=i):
              dstv = dst_l[i][pl.ds(ch, 16)]
              msk = dstv == 0
              srcv = src_l[i][pl.ds(ch, 16)]
              pos = cntv + plsc.cumsum(jnp.where(msk, 1, 0)) - 1
              plsc.store_scatter(comp_l[i], [pos], srcv, mask=msk)
              return cntv + plsc.all_reduce_population_count(msk)

          cntv = scan_loop
          plsc.store_scatter(comp_l[i], [cntv + iota], zeros16i)
          cnts.append(cntv[0])

          idxv = comp_l[i][pl.ds(0, 16)]
          nbids = plsc.load_gather(nb_l[i], [idxv])
          pltpu.async_copy(emb_hbm.at[nbids], rows_ch_l[i], sem_g[i])

      obuf = zerof
      for i in range(PAIRS_PER_W):
          cnt = cnts[i]
          pltpu.make_async_copy(emb_hbm.at[zeros16i], rows_ch_l[i],
                                sem_g[i]).wait()

          ad = []
          for h in range(HEADS):
              acc = zerof
              for q in range(HID // 16):
                  f0c = rows_ch_l[i][0, pl.ds(q * 16, 16)]
                  cc = plsc.load_gather(
                      c_v, [q * 16 + iota,
                            jnp.full((16,), 4 + h, jnp.int32)])
                  acc = acc + f0c * cc
              ad.append(jnp.sum(acc))

          nchunks = lax.shift_right_logical(cnt + 15, 4)

          def chunk_body(ch, st, i=i, cnt=cnt, ad=ad):
              @pl.when(ch > 0)
              def _():
                  idxv = comp_l[i][pl.ds(ch * 16, 16)]
                  nbids = plsc.load_gather(nb_l[i], [idxv])
                  pltpu.async_copy(emb_hbm.at[nbids], rows_ch_l[i],
                                   sem_g[i]).wait()

              valid = (ch * 16 + iota) < cnt

              def kbody(k, accs):
                  kv = jnp.full((16,), k, jnp.int32)
                  col = plsc.load_gather(rows_ch_l[i], [iota, kv])
                  ck = c_v[k]
                  new = []
                  for h in range(HEADS):
                      new.append(accs[h] + col * ck[h])
                  for h in range(HEADS):
                      new.append(accs[4 + h] + col * ck[8 + h])
                  return tuple(new)

              accs = lax.fori_loop(0, HID, kbody, (zerof,) * 8)

              out_st = []
              for h in range(HEADS):
                  m_h, den_h, s_h = st[h], st[4 + h], st[8 + h]
                  x = accs[h] + ad[h]
                  ev = jnp.where(x > 0, x, 0.2 * x)
                  evm = jnp.where(valid, ev, NEG)
                  mnew = jnp.maximum(m_h, jnp.max(evm))
                  exv = jnp.exp(evm - mnew)
                  oldsc = jnp.max(jnp.exp(jnp.full((16,), m_h - mnew)))
                  out_st.append(mnew)
                  out_st.append(den_h * oldsc + jnp.sum(exv))
                  out_st.append(s_h * oldsc + jnp.sum(exv * accs[4 + h]))
              return (out_st[0], out_st[3], out_st[6], out_st[9],
                      out_st[1], out_st[4], out_st[7], out_st[10],
                      out_st[2], out_st[5], out_st[8], out_st[11])

          init = (NEG,) * 4 + (0.0,) * 8
          st = lax.fori_loop(0, nchunks, chunk_body,
                             tuple(jnp.float32(v) for v in init))
          sv, dv = zerof, jnp.ones((16,), jnp.float32)
          for h in range(HEADS):
              sv = jnp.where(iota == h, st[8 + h], sv)
              dv = jnp.where(iota == h, st[4 + h] + 1e-16, dv)
          contrib = jnp.sum(sv / dv) * (1.0 / HEADS)
          obuf = jnp.where(iota == i, contrib, obuf)

      outb_v[...] = obuf

    @pl.when(g == 0)
    def _():
        run_graph(nb1_hbm, adj1_hbm)

    @pl.when(g == 1)
    def _():
        run_graph(nb2_hbm, adj2_hbm)

    pltpu.sync_copy(outb_v, out_hbm.at[wid])


@jax.jit
def _run_sc(nb1, nb2, adj1, adj2, emb, c_all):
    mesh = plsc.VectorSubcoreMesh(core_axis_name="c", subcore_axis_name="s",
                                  num_cores=NC, num_subcores=NS)
    fn = functools.partial(
        pl.kernel,
        out_type=jax.ShapeDtypeStruct((NW, 16), jnp.float32),
        mesh=mesh,
        compiler_params=pltpu.CompilerParams(needs_layout_passes=False),
        scratch_types=[
            [pltpu.VMEM((E,), jnp.int32)] * PAIRS_PER_W,
            [pltpu.VMEM((E,), jnp.int32)] * PAIRS_PER_W,
            [pltpu.VMEM((E + 32,), jnp.int32)] * PAIRS_PER_W,
            [pltpu.VMEM((N,), jnp.int32)] * PAIRS_PER_W,
            pltpu.VMEM((HID, 16), jnp.float32),
            [pltpu.VMEM((16, HID), jnp.float32)] * PAIRS_PER_W,
            pltpu.VMEM((16,), jnp.float32),
            [pltpu.SemaphoreType.DMA] * PAIRS_PER_W,
            [pltpu.SemaphoreType.DMA] * PAIRS_PER_W,
        ],
    )(_sc_kernel_body)
    return fn(nb1, nb2, adj1, adj2, emb, c_all)


def kernel(neighbors_node1, neighbors_node2, adj1, adj2, emb, W1, att_src1,
           att_dst1, bias1, W2, att_src2, att_dst2, bias2, mlp_w, mlp_b):
    mw = mlp_w[0]

    eye = jnp.eye(HEADS, dtype=jnp.float32)

    def tmat(a_s, a_d, mseg):
        ts = (a_s[:, :, None] * eye[:, None, :]).reshape(HEADS * HID, HEADS)
        td = (a_d[:, :, None] * eye[:, None, :]).reshape(HEADS * HID, HEADS)
        tp = (mseg[None, :, None] * eye[:, None, :]).reshape(
            HEADS * HID, HEADS)
        return jnp.concatenate(
            [ts, td, tp, jnp.zeros((HEADS * HID, 4), jnp.float32)], axis=1)

    t_all = jnp.stack([tmat(att_src1, att_dst1, mw[:HID]),
                       tmat(att_src2, att_dst2, mw[HID:])])
    c_all = jnp.stack([W1, W2]) @ t_all
    out = _run_sc(neighbors_node1, neighbors_node2, adj1, adj2, emb, c_all)
    flat = out[:, :PAIRS_PER_W].reshape(2 * B)
    const = bias1 @ mw[:HID] + bias2 @ mw[HID:] + mlp_b[0]
    return flat[:B] + flat[B:] + const
